# baseline (speedup 1.0000x reference)
import functools

import jax
import jax.numpy as jnp
from jax.experimental import pallas as pl

EPS = 1e-12
NEG_BIG = -1e30


def _ema_batch_kernel(pt_row_ref, pt_col_ref, z_ref, out_ref):
    L = z_ref.shape[1]
    pt_row = pt_row_ref[0]
    pt_col = pt_col_ref[0]

    ld_row = jnp.log(jnp.maximum(1.0 - pt_row, EPS))
    ld_col = jnp.log(jnp.maximum(1.0 - pt_col, EPS))

    row_ids = jax.lax.broadcasted_iota(jnp.int32, (L, L), 0)
    col_ids = jax.lax.broadcasted_iota(jnp.int32, (L, L), 1)
    tril = row_ids >= col_ids

    mask_le = (row_ids <= col_ids).astype(jnp.float32)
    mask_ge = tril.astype(jnp.float32)
    S_row = jax.lax.dot_general(
        ld_row, mask_le, (((1,), (0,)), ((), ())),
        precision=jax.lax.Precision.HIGHEST,
        preferred_element_type=jnp.float32)
    S_col = jax.lax.dot_general(
        mask_ge, ld_col, (((1,), (0,)), ((), ())),
        precision=jax.lax.Precision.HIGHEST,
        preferred_element_type=jnp.float32)

    delta = S_col - S_row
    delta = jnp.where(tril, delta, NEG_BIG)
    W = jnp.exp(delta) * pt_row

    out_ref[0] = jax.lax.dot_general(
        W, z_ref[0], (((1,), (0,)), ((), ())),
        preferred_element_type=jnp.float32)


@jax.jit
def kernel(z, pt):
    B, L, D = z.shape
    pt_row3 = pt[:, None, :]
    pt_col3 = pt[:, :, None]
    grid = (B,)
    out = pl.pallas_call(
        _ema_batch_kernel,
        grid=grid,
        in_specs=[
            pl.BlockSpec((1, 1, L), lambda b: (b, 0, 0)),
            pl.BlockSpec((1, L, 1), lambda b: (b, 0, 0)),
            pl.BlockSpec((1, L, D), lambda b: (b, 0, 0)),
        ],
        out_specs=pl.BlockSpec((1, L, D), lambda b: (b, 0, 0)),
        out_shape=jax.ShapeDtypeStruct((B, L, D), jnp.float32),
    )(pt_row3, pt_col3, z)
    return out


# speedup vs baseline: 1.4195x; 1.4195x over previous
import jax
import jax.numpy as jnp
from jax.experimental import pallas as pl

EPS = 1e-12
NEG_BIG = -1e30
CHUNK = 128


def _cumsum_col(x):
    L = x.shape[0]
    s = x
    k = 1
    while k < L:
        s = s + jnp.concatenate([jnp.zeros((k, 1), jnp.float32), s[:-k]], axis=0)
        k *= 2
    return s


def _cumsum_row(x):
    L = x.shape[1]
    s = x
    k = 1
    while k < L:
        s = s + jnp.concatenate([jnp.zeros((1, k), jnp.float32), s[:, :-k]], axis=1)
        k *= 2
    return s


def _ema_batch_kernel(pt_row_ref, pt_col_ref, z_ref, out_ref):
    L, D = z_ref.shape[1], z_ref.shape[2]
    T = CHUNK
    pt_row = pt_row_ref[0]
    pt_col = pt_col_ref[0]

    ld_row = jnp.log(jnp.maximum(1.0 - pt_row, EPS))
    ld_col = jnp.log(jnp.maximum(1.0 - pt_col, EPS))
    s_row = _cumsum_row(ld_row)
    s_col = _cumsum_col(ld_col)

    rid = jax.lax.broadcasted_iota(jnp.int32, (T, T), 0)
    cid = jax.lax.broadcasted_iota(jnp.int32, (T, T), 1)
    tril = rid >= cid

    z = z_ref[0]
    c = jnp.zeros((1, D), jnp.float32)
    for k in range(L // T):
        r = k * T
        sc = s_col[r:r + T]
        sr = s_row[:, r:r + T]
        ptr = pt_row[:, r:r + T]
        delta = jnp.where(tril, sc - sr, NEG_BIG)
        w = jnp.exp(delta) * ptr
        f = jnp.exp(sc - sc[0:1])
        out_c = jax.lax.dot_general(
            w, z[r:r + T], (((1,), (0,)), ((), ())),
            preferred_element_type=jnp.float32) + f * c
        out_ref[0, r:r + T, :] = out_c
        if k + 1 < L // T:
            dec_next = jnp.maximum(1.0 - pt_col[r + T:r + T + 1], EPS)
            c = dec_next * out_c[T - 1:T]


@jax.jit
def kernel(z, pt):
    B, L, D = z.shape
    pt_row3 = pt[:, None, :]
    pt_col3 = pt[:, :, None]
    out = pl.pallas_call(
        _ema_batch_kernel,
        grid=(B,),
        in_specs=[
            pl.BlockSpec((1, 1, L), lambda b: (b, 0, 0)),
            pl.BlockSpec((1, L, 1), lambda b: (b, 0, 0)),
            pl.BlockSpec((1, L, D), lambda b: (b, 0, 0)),
        ],
        out_specs=pl.BlockSpec((1, L, D), lambda b: (b, 0, 0)),
        out_shape=jax.ShapeDtypeStruct((B, L, D), jnp.float32),
    )(pt_row3, pt_col3, z)
    return out


# speedup vs baseline: 1.6287x; 1.1474x over previous
import jax
import jax.numpy as jnp
from jax.experimental import pallas as pl

EPS = 1e-12
NEG_BIG = -1e30
CHUNK = 128


def _ema_batch_kernel(ptr2_ref, ptc2_ref, z_ref, out_ref):
    L, D = z_ref.shape[1], z_ref.shape[2]
    T = CHUNK
    NC = L // T
    ptr2 = ptr2_ref[0]
    ptc2 = ptc2_ref[0]

    ldr = jnp.log(jnp.maximum(1.0 - ptr2, EPS))
    ldc = jnp.log(jnp.maximum(1.0 - ptc2, EPS))

    rid = jax.lax.broadcasted_iota(jnp.int32, (T, T), 0)
    cid = jax.lax.broadcasted_iota(jnp.int32, (T, T), 1)
    tril = rid >= cid

    tril_f = tril.astype(jnp.float32)
    triu_f = (rid <= cid).astype(jnp.float32)
    u_col = jax.lax.dot_general(
        tril_f, ldc, (((1,), (0,)), ((), ())),
        precision=jax.lax.Precision.HIGHEST,
        preferred_element_type=jnp.float32)
    u_row = jax.lax.dot_general(
        ldr, triu_f, (((1,), (0,)), ((), ())),
        precision=jax.lax.Precision.HIGHEST,
        preferred_element_type=jnp.float32)

    z = z_ref[0]
    c = jnp.zeros((1, D), jnp.float32)
    for k in range(NC):
        r = k * T
        sc = u_col[:, k:k + 1]
        sr = u_row[k:k + 1, :]
        ptr = ptr2[k:k + 1, :]
        delta = jnp.where(tril, sc - sr, NEG_BIG)
        w = jnp.exp(delta) * ptr
        f = jnp.exp(sc - sc[0:1, :])
        out_c = jax.lax.dot_general(
            w, z[r:r + T], (((1,), (0,)), ((), ())),
            preferred_element_type=jnp.float32) + f * c
        out_ref[0, r:r + T, :] = out_c
        if k + 1 < NC:
            dec_next = jnp.maximum(1.0 - ptc2[0:1, k + 1:k + 2], EPS)
            c = dec_next * out_c[T - 1:T]


@jax.jit
def kernel(z, pt):
    B, L, D = z.shape
    T = CHUNK
    NC = L // T
    pt_row2 = pt.reshape(B, NC, T)
    pt_col2 = jnp.swapaxes(pt_row2, 1, 2)
    out = pl.pallas_call(
        _ema_batch_kernel,
        grid=(B,),
        in_specs=[
            pl.BlockSpec((1, NC, T), lambda b: (b, 0, 0)),
            pl.BlockSpec((1, T, NC), lambda b: (b, 0, 0)),
            pl.BlockSpec((1, L, D), lambda b: (b, 0, 0)),
        ],
        out_specs=pl.BlockSpec((1, L, D), lambda b: (b, 0, 0)),
        out_shape=jax.ShapeDtypeStruct((B, L, D), jnp.float32),
    )(pt_row2, pt_col2, z)
    return out
